# Initial kernel scaffold; baseline (speedup 1.0000x reference)
import jax
import jax.numpy as jnp
from jax.experimental import pallas as pl


def kernel(x, edge_index, edge_attr, batch_index, W, b, gn_weight, gn_bias, mean_scale):
    raise NotImplementedError("write your pallas kernel here")


# speedup vs baseline: 7.1346x; 7.1346x over previous
import functools

import jax
import jax.numpy as jnp
from jax import lax
from jax.experimental import pallas as pl
from jax.experimental.pallas import tpu as pltpu
from jax.experimental.pallas import tpu_sc as plsc

_N = 10000
_D = 128
_E = 320000
_G = 64
_EPS = 1e-5

_NC = 2
_NS = 16
_NW = _NC * _NS
_C = 256
_TCHUNK = _E // _C
_ITER = -(-_TCHUNK // _NW)
_NGRP = _C // 16
_R = 200
_NRCH = _N // _R


def _sc_aggregate_fn():
    mesh = plsc.VectorSubcoreMesh(core_axis_name="c", subcore_axis_name="s")

    @functools.partial(
        pl.kernel,
        mesh=mesh,
        out_type=jax.ShapeDtypeStruct((_NC * _N, _D), jnp.float32),
        scratch_types=[
            pltpu.VMEM((_C,), jnp.int32),
            pltpu.VMEM((_C,), jnp.int32),
            pltpu.VMEM((_C,), jnp.float32),
            pltpu.VMEM((_C, _D), jnp.float32),
            pltpu.VMEM_SHARED((_N, _D), jnp.float32),
        ],
    )
    def body(x_hbm, src_hbm, dst_hbm, ea_hbm, out_hbm,
             src_v, dst_v, ea_v, rows_v, acc_sh):
        c = lax.axis_index("c")
        s = lax.axis_index("s")
        w = s * _NC + c

        zero16 = jnp.zeros((16,), jnp.float32)

        def zrow(r, carry):
            for j in range(_D // 16):
                rows_v[r, pl.ds(j * 16, 16)] = zero16
            return carry
        lax.fori_loop(0, _C, zrow, 0)

        for k in range(-(-_NRCH // _NS)):
            j = s + _NS * k

            @pl.when(j < _NRCH)
            def _():
                pltpu.sync_copy(rows_v.at[pl.ds(0, _R)],
                                acc_sh.at[pl.ds(j * _R, _R)])

        plsc.subcore_barrier()

        dnums = lax.GatherDimensionNumbers(
            offset_dims=(), collapsed_slice_dims=(0,), start_index_map=(0,))

        def do_chunk(cid):
            base = pl.multiple_of(cid * _C, 8)
            pltpu.sync_copy(src_hbm.at[pl.ds(base, _C)], src_v)
            pltpu.sync_copy(dst_hbm.at[pl.ds(base, _C)], dst_v)
            pltpu.sync_copy(ea_hbm.at[pl.ds(base, _C)], ea_v)
            pltpu.sync_copy(x_hbm.at[src_v], rows_v)

            def group(g, gcarry):
                ea_vec = ea_v[pl.ds(g * 16, 16)]
                for j in range(16):
                    e = g * 16 + j
                    be = lax.gather(
                        ea_vec, jnp.full((16, 1), j, jnp.int32), dnums, (1,),
                        mode=lax.GatherScatterMode.PROMISE_IN_BOUNDS)
                    for f in range(_D // 16):
                        sl = pl.ds(f * 16, 16)
                        rows_v[e, sl] = rows_v[e, sl] * be
                return gcarry
            lax.fori_loop(0, _NGRP, group, 0)

            pltpu.sync_copy(rows_v, acc_sh.at[dst_v], add=True)

        def chunk(t, carry):
            cid = w + _NW * t

            @pl.when(cid < _TCHUNK)
            def _():
                do_chunk(cid)
            return carry
        lax.fori_loop(0, _ITER, chunk, 0)

        plsc.subcore_barrier()

        for k in range(-(-_NRCH // _NS)):
            j = s + _NS * k

            @pl.when(j < _NRCH)
            def _():
                pltpu.sync_copy(acc_sh.at[pl.ds(j * _R, _R)],
                                out_hbm.at[pl.ds(c * _N + j * _R, _R)])

    return body


def _tc_norm(parts, W, b, batch, gnw, gnb, ms):
    def body(parts_ref, w_ref, b_ref, batch_ref, gnw_ref, gnb_ref, ms_ref,
             out_ref):
        aggx = parts_ref[0] + parts_ref[1]
        z = jnp.dot(aggx, w_ref[...], preferred_element_type=jnp.float32)
        z = jnp.maximum(z + b_ref[...], 0.0)

        g_iota = lax.broadcasted_iota(jnp.int32, (_G, _N), 0)
        onehot = (batch_ref[...] == g_iota).astype(jnp.float32)
        cnt = jnp.maximum(jnp.sum(onehot, axis=1), 1.0)[:, None]

        sums = lax.dot_general(onehot, z, (((1,), (0,)), ((), ())),
                               preferred_element_type=jnp.float32)
        mean = sums / cnt
        meanb = lax.dot_general(onehot, mean * ms_ref[...],
                                (((0,), (0,)), ((), ())),
                                preferred_element_type=jnp.float32)
        centered = z - meanb
        var = lax.dot_general(onehot, centered * centered,
                              (((1,), (0,)), ((), ())),
                              preferred_element_type=jnp.float32) / cnt
        rstd = 1.0 / jnp.sqrt(var + _EPS)
        rstdb = lax.dot_general(onehot, rstd, (((0,), (0,)), ((), ())),
                                preferred_element_type=jnp.float32)
        out_ref[...] = gnw_ref[...] * centered * rstdb + gnb_ref[...]

    return pl.pallas_call(
        body,
        out_shape=jax.ShapeDtypeStruct((_N, _D), jnp.float32),
    )(parts, W, b, batch, gnw, gnb, ms)


def kernel(x, edge_index, edge_attr, batch_index, W, b, gn_weight, gn_bias,
           mean_scale):
    src = edge_index[0]
    dst = edge_index[1]
    parts = _sc_aggregate_fn()(x, src, dst, edge_attr)
    parts = parts.reshape(_NC, _N, _D)
    return _tc_norm(parts, W, b.reshape(1, _D), batch_index.reshape(1, _N),
                    gn_weight.reshape(1, _D), gn_bias.reshape(1, _D),
                    mean_scale.reshape(1, _D))


# speedup vs baseline: 9.7924x; 1.3725x over previous
import functools

import jax
import jax.numpy as jnp
from jax import lax
from jax.experimental import pallas as pl
from jax.experimental.pallas import tpu as pltpu
from jax.experimental.pallas import tpu_sc as plsc

_N = 10000
_D = 128
_E = 320000
_G = 64
_EPS = 1e-5

_NC = 2
_NS = 16
_NW = _NC * _NS
_C = 128
_TCHUNK = _E // _C
_NGRP = _C // 16
_NBUF = 3
_KMAX = 27
_R = 80
_NRCH = _N // _R


def _sc_aggregate_fn():
    mesh = plsc.VectorSubcoreMesh(core_axis_name="c", subcore_axis_name="s")

    @functools.partial(
        pl.kernel,
        mesh=mesh,
        out_type=jax.ShapeDtypeStruct((_NC * _N, _D), jnp.float32),
        scratch_types=(
            [pltpu.VMEM((_C,), jnp.int32) for _ in range(_NBUF)]
            + [pltpu.VMEM((_C,), jnp.int32) for _ in range(_NBUF)]
            + [pltpu.VMEM((_C,), jnp.float32) for _ in range(_NBUF)]
            + [pltpu.VMEM((_C, _D), jnp.float32) for _ in range(_NBUF)]
            + [pltpu.VMEM_SHARED((_N, _D), jnp.float32)]
            + [pltpu.SemaphoreType.DMA for _ in range(3 * _NBUF)]
        ),
    )
    def body(x_hbm, src_hbm, dst_hbm, ea_hbm, out_hbm,
             src0, src1, src2, dst0, dst1, dst2, ea0, ea1, ea2,
             rows0, rows1, rows2, acc_sh,
             isem0, isem1, isem2, gsem0, gsem1, gsem2,
             ssem0, ssem1, ssem2):
        SRC = [src0, src1, src2]
        DST = [dst0, dst1, dst2]
        EA = [ea0, ea1, ea2]
        ROWS = [rows0, rows1, rows2]
        ISEM = [isem0, isem1, isem2]
        GSEM = [gsem0, gsem1, gsem2]
        SSEM = [ssem0, ssem1, ssem2]

        c = lax.axis_index("c")
        s = lax.axis_index("s")
        w = s * _NC + c

        zero16 = jnp.zeros((16,), jnp.float32)

        def zrow(r, carry):
            for j in range(_D // 16):
                rows0[r, pl.ds(j * 16, 16)] = zero16
            return carry
        lax.fori_loop(0, _R, zrow, 0)

        for k in range(-(-_NRCH // _NS)):
            j = s + _NS * k

            @pl.when(j < _NRCH)
            def _():
                pltpu.sync_copy(rows0.at[pl.ds(0, _R)],
                                acc_sh.at[pl.ds(j * _R, _R)])

        plsc.subcore_barrier()

        dnums = lax.GatherDimensionNumbers(
            offset_dims=(), collapsed_slice_dims=(0,), start_index_map=(0,))

        def idx_issue(b, cid):
            base = pl.multiple_of(cid * _C, 8)
            pltpu.async_copy(src_hbm.at[pl.ds(base, _C)], SRC[b], ISEM[b])
            pltpu.async_copy(dst_hbm.at[pl.ds(base, _C)], DST[b], ISEM[b])
            pltpu.async_copy(ea_hbm.at[pl.ds(base, _C)], EA[b], ISEM[b])

        def idx_wait(b):
            pltpu.make_async_copy(src_hbm.at[pl.ds(0, _C)], SRC[b],
                                  ISEM[b]).wait()
            pltpu.make_async_copy(dst_hbm.at[pl.ds(0, _C)], DST[b],
                                  ISEM[b]).wait()
            pltpu.make_async_copy(ea_hbm.at[pl.ds(0, _C)], EA[b],
                                  ISEM[b]).wait()

        def gather_issue(b):
            pltpu.async_copy(x_hbm.at[SRC[b]], ROWS[b], GSEM[b])

        def gather_wait(b):
            pltpu.make_async_copy(x_hbm.at[SRC[b]], ROWS[b], GSEM[b]).wait()

        def scatter_issue(b):
            pltpu.async_copy(ROWS[b], acc_sh.at[DST[b]], SSEM[b], add=True)

        def scatter_wait(b):
            pltpu.make_async_copy(ROWS[b], acc_sh.at[DST[b]], SSEM[b]).wait()

        def scale(b):
            def group(g, gcarry):
                ea_vec = EA[b][pl.ds(g * 16, 16)]
                for j in range(16):
                    e = g * 16 + j
                    be = lax.gather(
                        ea_vec, jnp.full((16, 1), j, jnp.int32), dnums, (1,),
                        mode=lax.GatherScatterMode.PROMISE_IN_BOUNDS)
                    for f in range(_D // 16):
                        sl = pl.ds(f * 16, 16)
                        ROWS[b][e, sl] = ROWS[b][e, sl] * be
                return gcarry
            lax.fori_loop(0, _NGRP, group, 0)

        idx_issue(0, w)
        idx_wait(0)
        gather_issue(0)
        idx_issue(1, w + _NW)

        def pipe(k, carry):
            for jj in range(_NBUF):
                t = _NBUF * k + jj
                p = jj
                q = (jj + 1) % _NBUF
                cid = w + _NW * t
                cid1 = cid + _NW

                @pl.when((t >= 2) & (cid - 2 * _NW < _TCHUNK) & (t - 2 >= 0))
                def _():
                    scatter_wait(q)

                @pl.when((t >= 1) & (cid1 < _TCHUNK))
                def _():
                    idx_issue(q, cid1)

                @pl.when(cid < _TCHUNK)
                def _():
                    gather_wait(p)
                    scale(p)
                    scatter_issue(p)

                @pl.when(cid1 < _TCHUNK)
                def _():
                    idx_wait(q)
                    gather_issue(q)
            return carry
        lax.fori_loop(0, _KMAX, pipe, 0)

        plsc.subcore_barrier()

        for k in range(-(-_NRCH // _NS)):
            j = s + _NS * k

            @pl.when(j < _NRCH)
            def _():
                pltpu.sync_copy(acc_sh.at[pl.ds(j * _R, _R)],
                                out_hbm.at[pl.ds(c * _N + j * _R, _R)])

    return body


def _tc_norm(parts, W, b, batch, gnw, gnb, ms):
    def body(parts_ref, w_ref, b_ref, batch_ref, gnw_ref, gnb_ref, ms_ref,
             out_ref):
        aggx = parts_ref[0] + parts_ref[1]
        z = jnp.dot(aggx, w_ref[...], preferred_element_type=jnp.float32)
        z = jnp.maximum(z + b_ref[...], 0.0)

        g_iota = lax.broadcasted_iota(jnp.int32, (_G, _N), 0)
        onehot = (batch_ref[...] == g_iota).astype(jnp.float32)
        cnt = jnp.maximum(jnp.sum(onehot, axis=1), 1.0)[:, None]

        sums = lax.dot_general(onehot, z, (((1,), (0,)), ((), ())),
                               preferred_element_type=jnp.float32)
        mean = sums / cnt
        meanb = lax.dot_general(onehot, mean * ms_ref[...],
                                (((0,), (0,)), ((), ())),
                                preferred_element_type=jnp.float32)
        centered = z - meanb
        var = lax.dot_general(onehot, centered * centered,
                              (((1,), (0,)), ((), ())),
                              preferred_element_type=jnp.float32) / cnt
        rstd = 1.0 / jnp.sqrt(var + _EPS)
        rstdb = lax.dot_general(onehot, rstd, (((0,), (0,)), ((), ())),
                                preferred_element_type=jnp.float32)
        out_ref[...] = gnw_ref[...] * centered * rstdb + gnb_ref[...]

    return pl.pallas_call(
        body,
        out_shape=jax.ShapeDtypeStruct((_N, _D), jnp.float32),
    )(parts, W, b, batch, gnw, gnb, ms)


def kernel(x, edge_index, edge_attr, batch_index, W, b, gn_weight, gn_bias,
           mean_scale):
    src = edge_index[0]
    dst = edge_index[1]
    parts = _sc_aggregate_fn()(x, src, dst, edge_attr)
    parts = parts.reshape(_NC, _N, _D)
    return _tc_norm(parts, W, b.reshape(1, _D), batch_index.reshape(1, _N),
                    gn_weight.reshape(1, _D), gn_bias.reshape(1, _D),
                    mean_scale.reshape(1, _D))


# speedup vs baseline: 13.4234x; 1.3708x over previous
import functools

import jax
import jax.numpy as jnp
from jax import lax
from jax.experimental import pallas as pl
from jax.experimental.pallas import tpu as pltpu
from jax.experimental.pallas import tpu_sc as plsc

_N = 10000
_D = 128
_E = 320000
_G = 64
_EPS = 1e-5

_NC = 2
_NS = 16
_NW = _NC * _NS
_C = 128
_TCHUNK = _E // _C
_NGRP = _C // 16
_NBUF = 3
_KMAX = 27
_R = 80
_NRCH = _N // _R


def _sc_aggregate_fn():
    mesh = plsc.VectorSubcoreMesh(core_axis_name="c", subcore_axis_name="s")

    @functools.partial(
        pl.kernel,
        mesh=mesh,
        out_type=jax.ShapeDtypeStruct((_NC * _N, _D), jnp.float32),
        scratch_types=(
            [pltpu.VMEM((_C,), jnp.int32) for _ in range(_NBUF)]
            + [pltpu.VMEM((_C,), jnp.int32) for _ in range(_NBUF)]
            + [pltpu.VMEM((_C,), jnp.float32) for _ in range(_NBUF)]
            + [pltpu.VMEM((_C, _D), jnp.float32) for _ in range(_NBUF)]
            + [pltpu.VMEM_SHARED((_N, _D), jnp.float32)]
            + [pltpu.SemaphoreType.DMA for _ in range(4 * _NBUF)]
        ),
    )
    def body(x_hbm, src_hbm, dst_hbm, ea_hbm, out_hbm,
             src0, src1, src2, dst0, dst1, dst2, ea0, ea1, ea2,
             rows0, rows1, rows2, acc_sh,
             isem0, isem1, isem2, dsem0, dsem1, dsem2,
             gsem0, gsem1, gsem2, ssem0, ssem1, ssem2):
        SRC = [src0, src1, src2]
        DST = [dst0, dst1, dst2]
        EA = [ea0, ea1, ea2]
        ROWS = [rows0, rows1, rows2]
        ISEM = [isem0, isem1, isem2]
        DSEM = [dsem0, dsem1, dsem2]
        GSEM = [gsem0, gsem1, gsem2]
        SSEM = [ssem0, ssem1, ssem2]

        c = lax.axis_index("c")
        s = lax.axis_index("s")
        w = s * _NC + c

        zero16 = jnp.zeros((16,), jnp.float32)

        def zrow(r, carry):
            for j in range(_D // 16):
                rows0[r, pl.ds(j * 16, 16)] = zero16
            return carry
        lax.fori_loop(0, _R, zrow, 0)

        for k in range(-(-_NRCH // _NS)):
            j = s + _NS * k

            @pl.when(j < _NRCH)
            def _():
                pltpu.sync_copy(rows0.at[pl.ds(0, _R)],
                                acc_sh.at[pl.ds(j * _R, _R)])

        plsc.subcore_barrier()

        dnums = lax.GatherDimensionNumbers(
            offset_dims=(), collapsed_slice_dims=(0,), start_index_map=(0,))

        def srcea_issue(b, cid):
            base = pl.multiple_of(cid * _C, 8)
            pltpu.async_copy(src_hbm.at[pl.ds(base, _C)], SRC[b], ISEM[b])
            pltpu.async_copy(ea_hbm.at[pl.ds(base, _C)], EA[b], ISEM[b])

        def srcea_wait(b):
            pltpu.make_async_copy(src_hbm.at[pl.ds(0, _C)], SRC[b],
                                  ISEM[b]).wait()
            pltpu.make_async_copy(ea_hbm.at[pl.ds(0, _C)], EA[b],
                                  ISEM[b]).wait()

        def dst_issue(b, cid):
            base = pl.multiple_of(cid * _C, 8)
            pltpu.async_copy(dst_hbm.at[pl.ds(base, _C)], DST[b], DSEM[b])

        def dst_wait(b):
            pltpu.make_async_copy(dst_hbm.at[pl.ds(0, _C)], DST[b],
                                  DSEM[b]).wait()

        def gather_issue(b):
            pltpu.async_copy(x_hbm.at[SRC[b]], ROWS[b], GSEM[b])

        def gather_wait(b):
            pltpu.make_async_copy(x_hbm.at[SRC[b]], ROWS[b], GSEM[b]).wait()

        def scatter_issue(b):
            pltpu.async_copy(ROWS[b], acc_sh.at[DST[b]], SSEM[b], add=True)

        def scatter_wait(b):
            pltpu.make_async_copy(ROWS[b], acc_sh.at[DST[b]], SSEM[b]).wait()

        def scale(b):
            def group(g, gcarry):
                ea_vec = EA[b][pl.ds(g * 16, 16)]
                for j in range(16):
                    e = g * 16 + j
                    be = lax.gather(
                        ea_vec, jnp.full((16, 1), j, jnp.int32), dnums, (1,),
                        mode=lax.GatherScatterMode.PROMISE_IN_BOUNDS)
                    for f in range(_D // 16):
                        sl = pl.ds(f * 16, 16)
                        ROWS[b][e, sl] = ROWS[b][e, sl] * be
                return gcarry
            lax.fori_loop(0, _NGRP, group, 0)

        srcea_issue(0, w)
        srcea_wait(0)
        gather_issue(0)
        srcea_issue(1, w + _NW)
        dst_issue(0, w)

        def pipe(k, carry):
            for jj in range(_NBUF):
                t = _NBUF * k + jj
                p = jj
                q = (jj + 1) % _NBUF
                r = (jj + 2) % _NBUF
                cid = w + _NW * t
                cid1 = cid + _NW
                cid2 = cid + 2 * _NW

                @pl.when((t >= 2) & (cid - 2 * _NW < _TCHUNK))
                def _():
                    scatter_wait(q)

                @pl.when(cid1 < _TCHUNK)
                def _():
                    dst_issue(q, cid1)

                @pl.when(cid2 < _TCHUNK)
                def _():
                    srcea_issue(r, cid2)

                @pl.when(cid1 < _TCHUNK)
                def _():
                    srcea_wait(q)
                    gather_issue(q)

                @pl.when(cid < _TCHUNK)
                def _():
                    gather_wait(p)
                    scale(p)
                    dst_wait(p)
                    scatter_issue(p)
            return carry
        lax.fori_loop(0, _KMAX, pipe, 0)

        plsc.subcore_barrier()

        for k in range(-(-_NRCH // _NS)):
            j = s + _NS * k

            @pl.when(j < _NRCH)
            def _():
                pltpu.sync_copy(acc_sh.at[pl.ds(j * _R, _R)],
                                out_hbm.at[pl.ds(c * _N + j * _R, _R)])

    return body


def _tc_norm(parts, W, b, batch, gnw, gnb, ms):
    def body(parts_ref, w_ref, b_ref, batch_ref, gnw_ref, gnb_ref, ms_ref,
             out_ref):
        aggx = parts_ref[0] + parts_ref[1]
        z = jnp.dot(aggx, w_ref[...], preferred_element_type=jnp.float32)
        z = jnp.maximum(z + b_ref[...], 0.0)

        g_iota = lax.broadcasted_iota(jnp.int32, (_G, _N), 0)
        onehot = (batch_ref[...] == g_iota).astype(jnp.float32)
        cnt = jnp.maximum(jnp.sum(onehot, axis=1), 1.0)[:, None]

        sums = lax.dot_general(onehot, z, (((1,), (0,)), ((), ())),
                               preferred_element_type=jnp.float32)
        mean = sums / cnt
        meanb = lax.dot_general(onehot, mean * ms_ref[...],
                                (((0,), (0,)), ((), ())),
                                preferred_element_type=jnp.float32)
        centered = z - meanb
        var = lax.dot_general(onehot, centered * centered,
                              (((1,), (0,)), ((), ())),
                              preferred_element_type=jnp.float32) / cnt
        rstd = 1.0 / jnp.sqrt(var + _EPS)
        rstdb = lax.dot_general(onehot, rstd, (((0,), (0,)), ((), ())),
                                preferred_element_type=jnp.float32)
        out_ref[...] = gnw_ref[...] * centered * rstdb + gnb_ref[...]

    return pl.pallas_call(
        body,
        out_shape=jax.ShapeDtypeStruct((_N, _D), jnp.float32),
    )(parts, W, b, batch, gnw, gnb, ms)


def kernel(x, edge_index, edge_attr, batch_index, W, b, gn_weight, gn_bias,
           mean_scale):
    src = edge_index[0]
    dst = edge_index[1]
    parts = _sc_aggregate_fn()(x, src, dst, edge_attr)
    parts = parts.reshape(_NC, _N, _D)
    return _tc_norm(parts, W, b.reshape(1, _D), batch_index.reshape(1, _N),
                    gn_weight.reshape(1, _D), gn_bias.reshape(1, _D),
                    mean_scale.reshape(1, _D))
